# baseline (speedup 1.0000x reference)
import functools

import jax
import jax.numpy as jnp
from jax import lax
from jax.experimental import pallas as pl
from jax.experimental.pallas import tpu as pltpu
from jax.experimental.pallas import tpu_sc as plsc


_info = plsc.get_sparse_core_info()
_NC, _NS = _info.num_cores, _info.num_subcores
_NW = _NC * _NS

_CH = 128
_UPW = 200


def _transpose_table(W_T, W_tailT):
    R, V = W_T.shape
    BL = 3584
    NB = 999936 // BL
    N4 = BL // 4

    def body(i_ref, t_ref, o_ref):
        i = pl.program_id(0)

        @pl.when(i < NB)
        def _():
            o_ref[...] = (
                i_ref[...].reshape(R, N4, 4).transpose(1, 2, 0).reshape(N4, 128)
            )

        @pl.when(i == NB)
        def _():
            o_ref[...] = jnp.zeros((N4, 128), jnp.float32)
            o_ref[0:16, :] = (
                t_ref[...].reshape(R, 16, 4).transpose(1, 2, 0).reshape(16, 128)
            )

    return pl.pallas_call(
        body,
        grid=(NB + 1,),
        in_specs=[
            pl.BlockSpec((R, BL), lambda i: (0, jnp.minimum(i, NB - 1))),
            pl.BlockSpec((R, 64), lambda i: (0, 0)),
        ],
        out_specs=pl.BlockSpec((N4, 128), lambda i: (i, 0)),
        out_shape=jax.ShapeDtypeStruct((250016, 128), jnp.float32),
    )(W_T, W_tailT)


def _embed_gather(table, idx2):
    mesh = plsc.VectorSubcoreMesh(core_axis_name="c", subcore_axis_name="s")

    @functools.partial(
        pl.kernel,
        mesh=mesh,
        out_type=jax.ShapeDtypeStruct((200, 4, 32, 8, 128), jnp.float32),
        scratch_types=[
            pltpu.VMEM((_UPW, _CH), jnp.int32),
            pltpu.VMEM((2, _CH, 32), jnp.float32),
            pltpu.VMEM((2, 32, 128), jnp.float32),
            pltpu.SemaphoreType.DMA,
            pltpu.SemaphoreType.DMA,
            pltpu.SemaphoreType.DMA,
            pltpu.SemaphoreType.DMA,
        ],
        compiler_params=pltpu.CompilerParams(use_tc_tiling_on_sc=False, needs_layout_passes=False, disable_bounds_checks=True),
    )
    def k(table_hbm, idx_hbm, out_hbm, idx_v, rows_v, til_v, gsem_a, gsem_b, osem_a, osem_b):
        wid = lax.axis_index("s") * _NC + lax.axis_index("c")
        g0 = wid * _UPW
        pltpu.sync_copy(idx_hbm.at[pl.ds(g0, _UPW)], idx_v)

        lane = jnp.arange(16, dtype=jnp.int32)
        lanes8 = [lane + (16 * h) for h in range(8)]

        gsems = (gsem_a, gsem_b)
        osems = (osem_a, osem_b)

        def fire_gather(u, buf):
            pltpu.async_copy(table_hbm.at[idx_v.at[u]], rows_v.at[buf], gsems[buf])

        def wait_gather(u, buf):
            pltpu.make_async_copy(
                table_hbm.at[idx_v.at[u]], rows_v.at[buf], gsems[buf]
            ).wait()

        def permute(buf):
            rows = rows_v.at[buf]

            @plsc.parallel_loop(0, 32, unroll=4)
            def _(r):
                rcol = jnp.full((16,), 0, jnp.int32) + r
                for h in range(8):
                    vec = plsc.load_gather(rows, [lanes8[h], rcol])
                    til_v[buf, r, pl.ds(h * 16, 16)] = vec

        def fire_outs(j, bt, buf):
            for tr in range(4):
                pltpu.async_copy(
                    til_v.at[buf, pl.ds(tr * 8, 8)], out_hbm.at[j, tr, bt], osems[buf]
                )

        def wait_outs(j, bt, buf):
            for tr in range(4):
                pltpu.make_async_copy(
                    til_v.at[buf, pl.ds(tr * 8, 8)], out_hbm.at[j, tr, bt], osems[buf]
                ).wait()

        def unit_step(u, buf):
            g = g0 + u
            jt, rem = lax.div(g, 256), lax.rem(g, 256)
            bt, sr = lax.div(rem, 8), lax.rem(rem, 8)
            j = jt * 8 + sr

            @pl.when(u + 1 < _UPW)
            def _():
                fire_gather(u + 1, 1 - buf)

            wait_gather(u, buf)

            @pl.when(u >= 2)
            def _():
                wait_outs(j, bt, buf)

            permute(buf)
            fire_outs(j, bt, buf)

        fire_gather(0, 0)

        def body(u2, carry):
            unit_step(2 * u2, 0)
            unit_step(2 * u2 + 1, 1)
            return carry

        lax.fori_loop(0, _UPW // 2, body, 0, unroll=False)
        for b in range(2):
            for tr in range(4):
                pltpu.make_async_copy(
                    til_v.at[b, pl.ds(tr * 8, 8)], out_hbm.at[0, tr, 0], osems[b]
                ).wait()

    return k(table, idx2)


def kernel(x, W_E):
    B0, B1 = x.shape
    V, D = W_E.shape
    idx2 = (
        x.T.reshape(B1 // 8, 8, B0 // 128, 128)
        .transpose(0, 2, 1, 3)
        .reshape(B1 * B0 // 128, 128)
        .astype(jnp.int32)
    )
    tableL = _transpose_table(W_E.T, W_E[999936:].T).reshape(1000064, D)
    M = _embed_gather(tableL, idx2)
    return M.transpose(2, 4, 0, 1, 3).reshape(B0, B1, D)


# speedup vs baseline: 3.7009x; 3.7009x over previous
import functools

import jax
import jax.numpy as jnp
from jax import lax
from jax.experimental import pallas as pl
from jax.experimental.pallas import tpu as pltpu
from jax.experimental.pallas import tpu_sc as plsc


_info = plsc.get_sparse_core_info()
_NC, _NS = _info.num_cores, _info.num_subcores
_NW = _NC * _NS

_CH = 128
_UPW = 200


def _transpose_table(W_T, W_tail128):
    mesh = plsc.VectorSubcoreMesh(core_axis_name="c", subcore_axis_name="s")
    NU = 7813
    KPW = 244

    @functools.partial(
        pl.kernel,
        mesh=mesh,
        out_type=jax.ShapeDtypeStruct((250016, 128), jnp.float32),
        scratch_types=[
            pltpu.VMEM((2, 32, 128), jnp.float32),
            pltpu.VMEM((2, 32, 128), jnp.float32),
            pltpu.SemaphoreType.DMA,
            pltpu.SemaphoreType.DMA,
            pltpu.SemaphoreType.DMA,
            pltpu.SemaphoreType.DMA,
        ],
        compiler_params=pltpu.CompilerParams(
            use_tc_tiling_on_sc=True,
            needs_layout_passes=False,
            disable_bounds_checks=True,
        ),
    )
    def k(wt_hbm, tail_hbm, out_hbm, vin, vout, isem_a, isem_b, osem_a, osem_b):
        wid = lax.axis_index("s") * _NC + lax.axis_index("c")
        isems = (isem_a, isem_b)
        osems = (osem_a, osem_b)

        lane = jnp.arange(16, dtype=jnp.int32)
        rvecs = [(lane + 16 * h) % 32 for h in range(8)]
        qvecs = [(lane + 16 * h) // 32 for h in range(8)]

        def g_of(kk):
            return kk * _NW + wid

        def fire_in(kk, buf):
            g = g_of(kk)

            @pl.when(g < NU - 1)
            def _():
                pltpu.async_copy(
                    wt_hbm.at[:, pl.ds(pl.multiple_of(g * 128, 128), 128)],
                    vin.at[buf],
                    isems[buf],
                )

            @pl.when(g == NU - 1)
            def _():
                pltpu.async_copy(tail_hbm, vin.at[buf], isems[buf])

        def wait_in(buf):
            pltpu.make_async_copy(tail_hbm, vin.at[buf], isems[buf]).wait()

        def permute(buf):
            @plsc.parallel_loop(0, 32, unroll=4)
            def _(s):
                for h in range(8):
                    vec = plsc.load_gather(
                        vin.at[buf], [rvecs[h], qvecs[h] + 4 * s]
                    )
                    vout[buf, s, pl.ds(h * 16, 16)] = vec

        def fire_out(kk, buf):
            g = g_of(kk)
            pltpu.async_copy(
                vout.at[buf],
                out_hbm.at[pl.ds(pl.multiple_of(g * 32, 32), 32)],
                osems[buf],
            )

        def wait_out(buf):
            pltpu.make_async_copy(
                vout.at[buf], out_hbm.at[pl.ds(0, 32)], osems[buf]
            ).wait()

        def step(kk, buf):
            @pl.when(kk + 1 < KPW + 1)
            def _():
                @pl.when(g_of(kk + 1) < NU)
                def _():
                    fire_in(kk + 1, 1 - buf)

            @pl.when(g_of(kk) < NU)
            def _():
                wait_in(buf)

                @pl.when(kk >= 2)
                def _():
                    wait_out(buf)

                permute(buf)
                fire_out(kk, buf)

        fire_in(0, 0)

        def body(k2, carry):
            step(2 * k2, 0)
            step(2 * k2 + 1, 1)
            return carry

        n_rounds = KPW + 1 + ((KPW + 1) % 2)
        lax.fori_loop(0, n_rounds // 2, body, 0, unroll=False)
        for b in range(2):
            pltpu.make_async_copy(
                vout.at[b], out_hbm.at[pl.ds(0, 32)], osems[b]
            ).wait()

    return k(W_T, W_tail128)


def _embed_gather(table, idx2):
    mesh = plsc.VectorSubcoreMesh(core_axis_name="c", subcore_axis_name="s")

    @functools.partial(
        pl.kernel,
        mesh=mesh,
        out_type=jax.ShapeDtypeStruct((200, 4, 32, 8, 128), jnp.float32),
        scratch_types=[
            pltpu.VMEM((_UPW, _CH), jnp.int32),
            pltpu.VMEM((2, _CH, 32), jnp.float32),
            pltpu.VMEM((2, 32, 128), jnp.float32),
            pltpu.SemaphoreType.DMA,
            pltpu.SemaphoreType.DMA,
            pltpu.SemaphoreType.DMA,
            pltpu.SemaphoreType.DMA,
        ],
        compiler_params=pltpu.CompilerParams(use_tc_tiling_on_sc=False, needs_layout_passes=False, disable_bounds_checks=True),
    )
    def k(table_hbm, idx_hbm, out_hbm, idx_v, rows_v, til_v, gsem_a, gsem_b, osem_a, osem_b):
        wid = lax.axis_index("s") * _NC + lax.axis_index("c")
        g0 = wid * _UPW
        pltpu.sync_copy(idx_hbm.at[pl.ds(g0, _UPW)], idx_v)

        lane = jnp.arange(16, dtype=jnp.int32)
        lanes8 = [lane + (16 * h) for h in range(8)]

        gsems = (gsem_a, gsem_b)
        osems = (osem_a, osem_b)

        def fire_gather(u, buf):
            pltpu.async_copy(table_hbm.at[idx_v.at[u]], rows_v.at[buf], gsems[buf])

        def wait_gather(u, buf):
            pltpu.make_async_copy(
                table_hbm.at[idx_v.at[u]], rows_v.at[buf], gsems[buf]
            ).wait()

        def permute(buf):
            rows = rows_v.at[buf]

            @plsc.parallel_loop(0, 32, unroll=4)
            def _(r):
                rcol = jnp.full((16,), 0, jnp.int32) + r
                for h in range(8):
                    vec = plsc.load_gather(rows, [lanes8[h], rcol])
                    til_v[buf, r, pl.ds(h * 16, 16)] = vec

        def fire_outs(j, bt, buf):
            for tr in range(4):
                pltpu.async_copy(
                    til_v.at[buf, pl.ds(tr * 8, 8)], out_hbm.at[j, tr, bt], osems[buf]
                )

        def wait_outs(j, bt, buf):
            for tr in range(4):
                pltpu.make_async_copy(
                    til_v.at[buf, pl.ds(tr * 8, 8)], out_hbm.at[j, tr, bt], osems[buf]
                ).wait()

        def unit_step(u, buf):
            g = g0 + u
            jt, rem = lax.div(g, 256), lax.rem(g, 256)
            bt, sr = lax.div(rem, 8), lax.rem(rem, 8)
            j = jt * 8 + sr

            @pl.when(u + 1 < _UPW)
            def _():
                fire_gather(u + 1, 1 - buf)

            wait_gather(u, buf)

            @pl.when(u >= 2)
            def _():
                wait_outs(j, bt, buf)

            permute(buf)
            fire_outs(j, bt, buf)

        fire_gather(0, 0)

        def body(u2, carry):
            unit_step(2 * u2, 0)
            unit_step(2 * u2 + 1, 1)
            return carry

        lax.fori_loop(0, _UPW // 2, body, 0, unroll=False)
        for b in range(2):
            for tr in range(4):
                pltpu.make_async_copy(
                    til_v.at[b, pl.ds(tr * 8, 8)], out_hbm.at[0, tr, 0], osems[b]
                ).wait()

    return k(table, idx2)


def kernel(x, W_E):
    B0, B1 = x.shape
    V, D = W_E.shape
    idx2 = (
        x.T.reshape(B1 // 8, 8, B0 // 128, 128)
        .transpose(0, 2, 1, 3)
        .reshape(B1 * B0 // 128, 128)
        .astype(jnp.int32)
    )
    W_tail128 = jnp.pad(W_E[999936:], ((0, 64), (0, 0))).T
    tableL = _transpose_table(W_E.T, W_tail128).reshape(1000064, D)
    M = _embed_gather(tableL, idx2)
    return M.transpose(2, 4, 0, 1, 3).reshape(B0, B1, D)


# speedup vs baseline: 10.5793x; 2.8586x over previous
import functools

import jax
import jax.numpy as jnp
from jax import lax
from jax.experimental import pallas as pl
from jax.experimental.pallas import tpu as pltpu
from jax.experimental.pallas import tpu_sc as plsc


_info = plsc.get_sparse_core_info()
_NC, _NS = _info.num_cores, _info.num_subcores
_NW = _NC * _NS

_CH = 128
_UPW = 200


def _transpose_table(W_T, W_tail128):
    mesh = plsc.VectorSubcoreMesh(core_axis_name="c", subcore_axis_name="s")
    CW = 512
    NUF = 1952
    KPW = NUF // _NW

    @functools.partial(
        pl.kernel,
        mesh=mesh,
        out_type=jax.ShapeDtypeStruct((250016, 128), jnp.float32),
        scratch_types=[
            pltpu.VMEM((2, 32, CW + 1), jnp.float32),
            pltpu.VMEM((2, CW // 4, 128), jnp.float32),
            pltpu.SemaphoreType.DMA,
            pltpu.SemaphoreType.DMA,
            pltpu.SemaphoreType.DMA,
            pltpu.SemaphoreType.DMA,
        ],
        compiler_params=pltpu.CompilerParams(
            use_tc_tiling_on_sc=True,
            needs_layout_passes=False,
            disable_bounds_checks=True,
        ),
    )
    def k(wt_hbm, tail_hbm, out_hbm, vin, vout, isem_a, isem_b, osem_a, osem_b):
        wid = lax.axis_index("s") * _NC + lax.axis_index("c")
        isems = (isem_a, isem_b)
        osems = (osem_a, osem_b)

        lane = jnp.arange(16, dtype=jnp.int32)
        rvecs = [(lane + 16 * h) % 32 for h in range(8)]
        qvecs = [(lane + 16 * h) // 32 for h in range(8)]

        def g_of(kk):
            return kk * _NW + wid

        def fire_in(kk, buf):
            pltpu.async_copy(
                wt_hbm.at[:, pl.ds(pl.multiple_of(g_of(kk) * CW, 128), CW)],
                vin.at[buf, :, pl.ds(0, CW)],
                isems[buf],
            )

        def wait_in(buf):
            pltpu.make_async_copy(
                wt_hbm.at[:, pl.ds(0, CW)], vin.at[buf, :, pl.ds(0, CW)], isems[buf]
            ).wait()

        rowbase = lane // 4
        rot = [(lane + kk) % 16 for kk in range(16)]
        colk = [(lane % 4) * 32 + rot[kk] for kk in range(16)]

        def permute(buf, n_t):
            @plsc.parallel_loop(0, n_t, unroll=2)
            def _(t):
                cidx = lane + 16 * t
                rowv = rowbase + 4 * t
                for kk in range(16):
                    for half in (0, 16):
                        rv = rot[kk] if half == 0 else rot[kk] + 16
                        cv = colk[kk] if half == 0 else colk[kk] + 16
                        vec = plsc.load_gather(vin.at[buf], [rv, cidx])
                        plsc.store_scatter(vout.at[buf], [rowv, cv], vec)

        def fire_out(kk, buf):
            pltpu.async_copy(
                vout.at[buf],
                out_hbm.at[pl.ds(pl.multiple_of(g_of(kk) * (CW // 4), 32), CW // 4)],
                osems[buf],
            )

        def wait_out(buf):
            pltpu.make_async_copy(
                vout.at[buf], out_hbm.at[pl.ds(0, CW // 4)], osems[buf]
            ).wait()

        def step(kk, buf):
            @pl.when(kk + 1 < KPW)
            def _():
                fire_in(kk + 1, 1 - buf)

            wait_in(buf)

            @pl.when(kk >= 2)
            def _():
                wait_out(buf)

            permute(buf, CW // 16)
            fire_out(kk, buf)

        fire_in(0, 0)

        def body(k2, carry):
            step(2 * k2, 0)
            step(2 * k2 + 1, 1)
            return carry

        lax.fori_loop(0, KPW // 2, body, 0, unroll=False)
        step(KPW - 1, (KPW - 1) % 2)
        for b in range(2):
            pltpu.make_async_copy(
                vout.at[b], out_hbm.at[pl.ds(0, CW // 4)], osems[b]
            ).wait()

        @pl.when(wid == 0)
        def _():
            pltpu.sync_copy(wt_hbm.at[:, pl.ds(999424, CW)], vin.at[0, :, pl.ds(0, CW)])
            permute(0, CW // 16)
            pltpu.sync_copy(vout.at[0], out_hbm.at[pl.ds(249856, CW // 4)])

        @pl.when(wid == 1)
        def _():
            pltpu.sync_copy(tail_hbm, vin.at[1, :, pl.ds(0, 128)])
            permute(1, 8)
            pltpu.sync_copy(vout.at[1, pl.ds(0, 32)], out_hbm.at[pl.ds(249984, 32)])

    return k(W_T, W_tail128)


def _embed_gather(table, idx2):
    mesh = plsc.VectorSubcoreMesh(core_axis_name="c", subcore_axis_name="s")

    @functools.partial(
        pl.kernel,
        mesh=mesh,
        out_type=jax.ShapeDtypeStruct((200, 4, 32, 8, 128), jnp.float32),
        scratch_types=[
            pltpu.VMEM((_UPW, _CH), jnp.int32),
            pltpu.VMEM((2, _CH, 32), jnp.float32),
            pltpu.VMEM((2, 32, 129), jnp.float32),
            pltpu.SemaphoreType.DMA,
            pltpu.SemaphoreType.DMA,
            pltpu.SemaphoreType.DMA,
            pltpu.SemaphoreType.DMA,
        ],
        compiler_params=pltpu.CompilerParams(use_tc_tiling_on_sc=False, needs_layout_passes=False, disable_bounds_checks=True),
    )
    def k(table_hbm, idx_hbm, out_hbm, idx_v, rows_v, til_v, gsem_a, gsem_b, osem_a, osem_b):
        wid = lax.axis_index("s") * _NC + lax.axis_index("c")
        g0 = wid * _UPW
        pltpu.sync_copy(idx_hbm.at[pl.ds(g0, _UPW)], idx_v)

        lane = jnp.arange(16, dtype=jnp.int32)
        lanes8 = [lane + (16 * h) for h in range(8)]

        gsems = (gsem_a, gsem_b)
        osems = (osem_a, osem_b)

        def fire_gather(u, buf):
            pltpu.async_copy(table_hbm.at[idx_v.at[u]], rows_v.at[buf], gsems[buf])

        def wait_gather(u, buf):
            pltpu.make_async_copy(
                table_hbm.at[idx_v.at[u]], rows_v.at[buf], gsems[buf]
            ).wait()

        def permute(buf):
            @plsc.parallel_loop(0, 128, unroll=8)
            def _(lb):
                lbs = jnp.full((16,), 0, jnp.int32) + lb
                for h2 in range(2):
                    vec = rows_v[buf, lb, pl.ds(16 * h2, 16)]
                    plsc.store_scatter(til_v.at[buf], [lanes8[h2], lbs], vec)

        def fire_outs(j, bt, buf):
            for tr in range(4):
                pltpu.async_copy(
                    til_v.at[buf, pl.ds(tr * 8, 8), pl.ds(0, 128)], out_hbm.at[j, tr, bt], osems[buf]
                )

        def wait_outs(j, bt, buf):
            for tr in range(4):
                pltpu.make_async_copy(
                    til_v.at[buf, pl.ds(tr * 8, 8), pl.ds(0, 128)], out_hbm.at[j, tr, bt], osems[buf]
                ).wait()

        def unit_step(u, buf):
            g = g0 + u
            jt, rem = lax.div(g, 256), lax.rem(g, 256)
            bt, sr = lax.div(rem, 8), lax.rem(rem, 8)
            j = jt * 8 + sr

            @pl.when(u + 1 < _UPW)
            def _():
                fire_gather(u + 1, 1 - buf)

            wait_gather(u, buf)

            @pl.when(u >= 2)
            def _():
                wait_outs(j, bt, buf)

            permute(buf)
            fire_outs(j, bt, buf)

        fire_gather(0, 0)

        def body(u2, carry):
            unit_step(2 * u2, 0)
            unit_step(2 * u2 + 1, 1)
            return carry

        lax.fori_loop(0, _UPW // 2, body, 0, unroll=False)
        for b in range(2):
            for tr in range(4):
                pltpu.make_async_copy(
                    til_v.at[b, pl.ds(tr * 8, 8), pl.ds(0, 128)], out_hbm.at[0, tr, 0], osems[b]
                ).wait()

    return k(table, idx2)


def kernel(x, W_E):
    B0, B1 = x.shape
    V, D = W_E.shape
    idx2 = (
        x.T.reshape(B1 // 8, 8, B0 // 128, 128)
        .transpose(0, 2, 1, 3)
        .reshape(B1 * B0 // 128, 128)
        .astype(jnp.int32)
    )
    W_tail128 = jnp.pad(W_E[999936:], ((0, 64), (0, 0))).T
    tableL = _transpose_table(W_E.T, W_tail128).reshape(1000064, D)
    M = _embed_gather(tableL, idx2)
    return M.transpose(2, 4, 0, 1, 3).reshape(B0, B1, D)
